# Initial kernel scaffold; baseline (speedup 1.0000x reference)
import jax
import jax.numpy as jnp
from jax.experimental import pallas as pl


def kernel(x, edge_index, W, att_src, att_dst, bias):
    raise NotImplementedError("write your pallas kernel here")


# speedup vs baseline: 19.5543x; 19.5543x over previous
import functools

import jax
import jax.numpy as jnp
from jax import lax
from jax.experimental import pallas as pl
from jax.experimental.pallas import tpu as pltpu
from jax.experimental.pallas import tpu_sc as plsc

N_NODES = 10000
D_FEAT = 128
N_EDGES = 320000
HALF = 64
CW = 80
NC = 2
NS = 16
EDGES_PER_TILE = N_EDGES // NS
K = 80
NCHUNK = EDGES_PER_TILE // K
ROWS_PER_TILE = N_NODES // NS


def _pre_body(x_ref, w_ref, asrc_ref, adst_ref, hext_ref, as_ref, ad_ref):
    h = jnp.dot(x_ref[...], w_ref[...], preferred_element_type=jnp.float32)
    ones = jnp.ones((N_NODES, 1), jnp.float32)
    zeros = jnp.zeros((N_NODES, CW - HALF - 1), jnp.float32)
    lo = jnp.concatenate([h[:, :HALF], ones, zeros], axis=1)
    hi = jnp.concatenate([h[:, HALF:], ones, zeros], axis=1)
    hext_ref[...] = jnp.concatenate([lo, hi], axis=0)
    as_ref[...] = jnp.sum(h * asrc_ref[...][None, :], axis=1, keepdims=True)
    ad_ref[...] = jnp.sum(h * adst_ref[...][None, :], axis=1, keepdims=True)


_pre = pl.pallas_call(
    _pre_body,
    out_shape=(
        jax.ShapeDtypeStruct((NC * N_NODES, CW), jnp.float32),
        jax.ShapeDtypeStruct((N_NODES, 1), jnp.float32),
        jax.ShapeDtypeStruct((N_NODES, 1), jnp.float32),
    ),
)


def _sc_body(hext, a_s_h, a_d_h, src_h, dst_h, out_h,
             a_s_v, a_d_v, src_v, dst_v, rows_v, w_v, acc, sem):
    cid = lax.axis_index("c")
    sid = lax.axis_index("s")

    def zrow(i, _):
        for g in range(CW // 16):
            rows_v[i, pl.ds(g * 16, 16)] = jnp.zeros((16,), jnp.float32)
        return 0
    lax.fori_loop(0, K, zrow, 0)
    base = sid * ROWS_PER_TILE
    for q in range(7):
        pltpu.sync_copy(rows_v, acc.at[pl.ds(base + q * K, K)])
    pltpu.sync_copy(rows_v.at[pl.ds(0, ROWS_PER_TILE - 7 * K)],
                    acc.at[pl.ds(base + 7 * K, ROWS_PER_TILE - 7 * K)])

    pltpu.sync_copy(a_s_h, a_s_v)
    pltpu.sync_copy(a_d_h, a_d_v)
    pltpu.sync_copy(src_h.at[cid, sid], src_v)
    pltpu.sync_copy(dst_h.at[sid], dst_v)

    def vmax(ref):
        def body(i, m):
            return jnp.maximum(m, ref[pl.ds(i * 16, 16)])
        m = lax.fori_loop(0, N_NODES // 16, body,
                          jnp.full((16,), -jnp.inf, jnp.float32))
        return jnp.max(m)
    mtot = vmax(a_s_v) + vmax(a_d_v)
    m_bound = jnp.where(mtot > 0, mtot, mtot * 0.2)
    src_off = cid * N_NODES

    plsc.subcore_barrier()

    def chunk(j, _):
        cp = pltpu.async_copy(hext.at[src_v.at[j]], rows_v, sem)
        for g in range(K // 16):
            si = src_v[j, pl.ds(g * 16, 16)] - src_off
            di = dst_v[j, pl.ds(g * 16, 16)]
            e = plsc.load_gather(a_s_v, [si]) + plsc.load_gather(a_d_v, [di])
            e = jnp.where(e >= 0, e, e * 0.2)
            w_v[pl.ds(g * 16, 16)] = jnp.exp(e - m_bound)
        cp.wait()

        def srow(i, _):
            wi = w_v[pl.ds(i, 16)][0]
            for g in range(CW // 16):
                rows_v[i, pl.ds(g * 16, 16)] = rows_v[i, pl.ds(g * 16, 16)] * wi
            return 0
        lax.fori_loop(0, K, srow, 0)
        pltpu.sync_copy(rows_v, acc.at[dst_v.at[j]], add=True)
        return 0
    lax.fori_loop(0, NCHUNK, chunk, 0)

    plsc.subcore_barrier()
    pltpu.sync_copy(acc.at[pl.ds(base, ROWS_PER_TILE)],
                    out_h.at[cid, pl.ds(base, ROWS_PER_TILE)])


_sc_main = functools.partial(
    pl.kernel,
    out_type=jax.ShapeDtypeStruct((NC, N_NODES, CW), jnp.float32),
    mesh=plsc.VectorSubcoreMesh(core_axis_name="c", subcore_axis_name="s"),
    compiler_params=pltpu.CompilerParams(use_tc_tiling_on_sc=False,
                                         needs_layout_passes=False),
    scratch_types=[
        pltpu.VMEM((N_NODES,), jnp.float32),
        pltpu.VMEM((N_NODES,), jnp.float32),
        pltpu.VMEM((NCHUNK, K), jnp.int32),
        pltpu.VMEM((NCHUNK, K), jnp.int32),
        pltpu.VMEM((K, CW), jnp.float32),
        pltpu.VMEM((K + 16,), jnp.float32),
        pltpu.VMEM_SHARED((N_NODES, CW), jnp.float32),
        pltpu.SemaphoreType.DMA,
    ],
)(_sc_body)


def _post_body(part_ref, bias_ref, out_ref):
    numer = jnp.concatenate(
        [part_ref[0, :, :HALF], part_ref[1, :, :HALF]], axis=1)
    denom = part_ref[0, :, HALF:HALF + 1]
    out = numer / (denom + 1e-16) + bias_ref[...][None, :]
    out_ref[...] = jnp.maximum(out, 0.0)


_post = pl.pallas_call(
    _post_body,
    out_shape=jax.ShapeDtypeStruct((N_NODES, D_FEAT), jnp.float32),
)


@jax.jit
def kernel(x, edge_index, W, att_src, att_dst, bias):
    src = edge_index[0].astype(jnp.int32).reshape(NS, NCHUNK, K)
    dst = edge_index[1].astype(jnp.int32).reshape(NS, NCHUNK, K)
    src2 = jnp.stack([src, src + N_NODES])
    hext, a_s, a_d = _pre(x, W, att_src, att_dst)
    partials = _sc_main(hext, a_s.reshape(N_NODES), a_d.reshape(N_NODES),
                        src2, dst)
    return _post(partials, bias)
